# baseline (speedup 1.0000x reference)
import functools

import jax
import jax.numpy as jnp
from jax import lax
from jax.experimental import pallas as pl
from jax.experimental.pallas import tpu as pltpu
from jax.experimental.pallas import tpu_sc as plsc

_IGNORE = -1
_TOPK = 5


def _sc_count(yt_pad, yp2d, row0):
    info = plsc.get_sparse_core_info()
    nc, ns, L = info.num_cores, info.num_subcores, info.num_lanes
    nw = nc * ns
    B, vocab = yp2d.shape
    nv = vocab // L
    q = (B - row0) // nw
    mesh = plsc.VectorSubcoreMesh(core_axis_name="c", subcore_axis_name="s")

    @functools.partial(
        pl.kernel,
        mesh=mesh,
        out_type=jax.ShapeDtypeStruct((B - row0, L), jnp.float32),
        scratch_types=[
            pltpu.VMEM((L,), jnp.int32),
            pltpu.VMEM((vocab,), jnp.float32),
            pltpu.VMEM((L,), jnp.float32),
        ],
    )
    def k(yt_hbm, yp_hbm, out_hbm, t_v, row_v, a_v):
        wid = lax.axis_index("s") * nc + lax.axis_index("c")
        base = row0 + wid * q
        pltpu.sync_copy(yt_hbm.at[pl.ds(base, L)], t_v)
        iota = lax.iota(jnp.int32, L)
        for k_i in range(q):
            t = t_v[...][k_i]
            tcl = jnp.clip(t, 0, vocab - 1)
            pltpu.sync_copy(yp_hbm.at[base + k_i], row_v)
            jt = tcl // L
            off = tcl - jt * L
            xb = row_v[pl.ds(jt * L, L)]
            v_s = jnp.float32(0)
            for i in range(L):
                v_s = jnp.where(off == i, xb[i], v_s)
            v = jnp.full((L,), v_s, jnp.float32)

            def body(j, a):
                xv = row_v[pl.ds(j * L, L)]
                below = jnp.where(j < jt, jnp.float32(1), jnp.float32(0))
                return (a + jnp.where(xv > v, 1.0, 0.0)
                        + jnp.where(xv == v, below, jnp.float32(0)))

            colb = jt * L + iota
            acc = (jnp.where(xb == v, 1.0, 0.0)
                   * jnp.where(colb < tcl, 1.0, 0.0))
            acc = lax.fori_loop(0, nv, body, acc, unroll=8)
            a_v[...] = acc
            pltpu.sync_copy(a_v, out_hbm.at[base - row0 + k_i])

    return k(yt_pad, yp2d)


def _tc_body(nblocks, vocab, t_ref, yp_ref, out_ref, acc_ref):
    i = pl.program_id(0)

    @pl.when(i == 0)
    def _init():
        acc_ref[0] = 0.0
        acc_ref[1] = 0.0

    x = yp_ref[...]
    tt = t_ref[...]
    tcl = jnp.clip(tt, 0, vocab - 1)
    col = lax.broadcasted_iota(jnp.int32, x.shape, 1)
    vv = jnp.sum(jnp.where(col == tcl, x, 0.0),
                 axis=1, keepdims=True)
    beats = jnp.logical_or(
        x > vv, jnp.logical_and(x == vv, col < tcl))
    rank = jnp.sum(beats.astype(jnp.float32), axis=1, keepdims=True)
    valid = tt != _IGNORE
    hit = jnp.logical_and(rank < float(_TOPK), valid)
    acc_ref[0] += jnp.sum(hit.astype(jnp.float32))
    acc_ref[1] += jnp.sum(valid.astype(jnp.float32))

    @pl.when(i == nblocks - 1)
    def _fini():
        out_ref[0, 0] = acc_ref[0]
        out_ref[0, 1] = acc_ref[1]


def kernel(y_true, y_pred):
    B = y_true.size
    vocab = y_pred.shape[-1]
    yt = y_true.reshape(B).astype(jnp.int32)
    yp2d = y_pred.reshape(B, vocab)

    ROW0 = 768
    yt_pad = jnp.pad(yt, (0, 16))
    sc = _sc_count(yt_pad, yp2d, ROW0)

    R = 32
    nblocks = ROW0 // R
    tc = pl.pallas_call(
        functools.partial(_tc_body, nblocks, vocab),
        grid=(nblocks,),
        in_specs=[
            pl.BlockSpec((R, 1), lambda i: (i, 0)),
            pl.BlockSpec((R, vocab), lambda i: (i, 0)),
        ],
        out_specs=pl.BlockSpec((1, 2), lambda i: (0, 0),
                               memory_space=pltpu.SMEM),
        out_shape=jax.ShapeDtypeStruct((1, 2), jnp.float32),
        scratch_shapes=[pltpu.SMEM((2,), jnp.float32)],
    )(yt.reshape(B, 1), yp2d)

    rank_sc = jnp.sum(sc, axis=1)
    valid_sc = yt[ROW0:] != _IGNORE
    hit_sc = jnp.logical_and(rank_sc < float(_TOPK), valid_sc)
    hits = tc[0, 0] + jnp.sum(hit_sc.astype(jnp.float32))
    nval = tc[0, 1] + jnp.sum(valid_sc.astype(jnp.float32))
    return 100.0 * hits / nval


# speedup vs baseline: 1.0635x; 1.0635x over previous
import functools

import jax
import jax.numpy as jnp
from jax import lax
from jax.experimental import pallas as pl
from jax.experimental.pallas import tpu as pltpu
from jax.experimental.pallas import tpu_sc as plsc

_IGNORE = -1
_TOPK = 5


def _sc_count(yt_pad, yp2d, row0):
    info = plsc.get_sparse_core_info()
    nc, ns, L = info.num_cores, info.num_subcores, info.num_lanes
    nw = nc * ns
    B, vocab = yp2d.shape
    nv = vocab // L
    q = (B - row0) // nw
    mesh = plsc.VectorSubcoreMesh(core_axis_name="c", subcore_axis_name="s")

    @functools.partial(
        pl.kernel,
        mesh=mesh,
        out_type=jax.ShapeDtypeStruct((B - row0, L), jnp.float32),
        scratch_types=[
            pltpu.VMEM((L,), jnp.int32),
            pltpu.VMEM((vocab,), jnp.float32),
            pltpu.VMEM((L,), jnp.float32),
        ],
    )
    def k(yt_hbm, yp_hbm, out_hbm, t_v, row_v, a_v):
        wid = lax.axis_index("s") * nc + lax.axis_index("c")
        base = row0 + wid * q
        albase = (base // 8) * 8
        off0 = base - albase
        pltpu.sync_copy(yt_hbm.at[pl.ds(albase, L)], t_v)
        iota = lax.iota(jnp.int32, L)
        for k_i in range(q):
            idx = off0 + k_i
            t = jnp.int32(0)
            for i in range(L):
                t = jnp.where(idx == i, t_v[...][i], t)
            tcl = jnp.clip(t, 0, vocab - 1)
            pltpu.sync_copy(yp_hbm.at[base + k_i], row_v)
            jt = tcl // L
            off = tcl - jt * L
            xb = row_v[pl.ds(jt * L, L)]
            v_s = jnp.float32(0)
            for i in range(L):
                v_s = jnp.where(off == i, xb[i], v_s)
            v = jnp.full((L,), v_s, jnp.float32)

            def body(j, a):
                xv = row_v[pl.ds(j * L, L)]
                below = jnp.where(j < jt, jnp.float32(1), jnp.float32(0))
                return (a + jnp.where(xv > v, 1.0, 0.0)
                        + jnp.where(xv == v, below, jnp.float32(0)))

            colb = jt * L + iota
            acc = (jnp.where(xb == v, 1.0, 0.0)
                   * jnp.where(colb < tcl, 1.0, 0.0))
            acc = lax.fori_loop(0, nv, body, acc, unroll=8)
            a_v[...] = acc
            pltpu.sync_copy(a_v, out_hbm.at[base - row0 + k_i])

    return k(yt_pad, yp2d)


def _tc_body(nblocks, vocab, t_ref, yp_ref, out_ref, acc_ref):
    i = pl.program_id(0)

    @pl.when(i == 0)
    def _init():
        acc_ref[0] = 0.0
        acc_ref[1] = 0.0

    x = yp_ref[...]
    tt = t_ref[...]
    tcl = jnp.clip(tt, 0, vocab - 1)
    col = lax.broadcasted_iota(jnp.int32, x.shape, 1)
    vv = jnp.sum(jnp.where(col == tcl, x, 0.0),
                 axis=1, keepdims=True)
    beats = jnp.logical_or(
        x > vv, jnp.logical_and(x == vv, col < tcl))
    rank = jnp.sum(beats.astype(jnp.float32), axis=1, keepdims=True)
    valid = tt != _IGNORE
    hit = jnp.logical_and(rank < float(_TOPK), valid)
    acc_ref[0] += jnp.sum(hit.astype(jnp.float32))
    acc_ref[1] += jnp.sum(valid.astype(jnp.float32))

    @pl.when(i == nblocks - 1)
    def _fini():
        out_ref[0, 0] = acc_ref[0]
        out_ref[0, 1] = acc_ref[1]


def kernel(y_true, y_pred):
    B = y_true.size
    vocab = y_pred.shape[-1]
    yt = y_true.reshape(B).astype(jnp.int32)
    yp2d = y_pred.reshape(B, vocab)

    ROW0 = 896
    yt_pad = jnp.pad(yt, (0, 16))
    sc = _sc_count(yt_pad, yp2d, ROW0)

    R = 32
    nblocks = ROW0 // R
    tc = pl.pallas_call(
        functools.partial(_tc_body, nblocks, vocab),
        grid=(nblocks,),
        in_specs=[
            pl.BlockSpec((R, 1), lambda i: (i, 0)),
            pl.BlockSpec((R, vocab), lambda i: (i, 0)),
        ],
        out_specs=pl.BlockSpec((1, 2), lambda i: (0, 0),
                               memory_space=pltpu.SMEM),
        out_shape=jax.ShapeDtypeStruct((1, 2), jnp.float32),
        scratch_shapes=[pltpu.SMEM((2,), jnp.float32)],
    )(yt.reshape(B, 1), yp2d)

    rank_sc = jnp.sum(sc, axis=1)
    valid_sc = yt[ROW0:] != _IGNORE
    hit_sc = jnp.logical_and(rank_sc < float(_TOPK), valid_sc)
    hits = tc[0, 0] + jnp.sum(hit_sc.astype(jnp.float32))
    nval = tc[0, 1] + jnp.sum(valid_sc.astype(jnp.float32))
    return 100.0 * hits / nval
